# baseline (speedup 1.0000x reference)
import jax
import jax.numpy as jnp
from jax.experimental import pallas as pl
from jax.experimental.pallas import tpu as pltpu

_CH = 16
_EMB = 128
_HB = 8
_W = 384


def _fused_kernel(xt, xc, xb, fm, mle, mre, w1, b1, w2, b2, w3, b3, out):
    w = _W
    n = _HB * w
    nh = pl.num_programs(1)
    i = pl.program_id(1)
    top = xt[0] * jnp.where(i > 0, 1.0, 0.0)
    bot = xb[0] * jnp.where(i < nh - 1, 1.0, 0.0)
    xe = jnp.concatenate([top, xc[0], bot], axis=1)

    up = xe[:, w:w + n]
    mid = xe[:, 2 * w:2 * w + n]
    dn = xe[:, 3 * w:3 * w + n]
    ul = xe[:, w + 1:w + 1 + n]
    ur = xe[:, w - 1:w - 1 + n]
    cl = xe[:, 2 * w + 1:2 * w + 1 + n]
    cr = xe[:, 2 * w - 1:2 * w - 1 + n]
    dl = xe[:, 3 * w + 1:3 * w + 1 + n]
    dr = xe[:, 3 * w - 1:3 * w - 1 + n]
    ml = mle[...]
    mr = mre[...]

    gx = (ul + 2.0 * cl + dl) * ml - (ur + 2.0 * cr + dr) * mr
    gy = (dl - ul) * ml + (dr - ur) * mr + 2.0 * (dn - up)

    xa = xe[3:4, :]
    pmf = jnp.maximum(jnp.maximum(xa[:, :n + 2 * w], xa[:, w:n + 3 * w]),
                      xa[:, 2 * w:])
    pooled = jnp.maximum(
        jnp.maximum(pmf[:, w + 1:w + 1 + n] * ml, pmf[:, w - 1:w - 1 + n] * mr),
        pmf[:, w:w + n])
    act = jnp.where((pooled > 0.1) & (fm[0] != 0), 1.0, 0.0)

    f = jnp.concatenate([mid, gx, gy], axis=0)
    prec = jax.lax.Precision.DEFAULT
    h1 = jnp.maximum(
        jnp.dot(w1[...], f, preferred_element_type=jnp.float32, precision=prec)
        + b1[...], 0.0)
    h2 = jnp.maximum(
        jnp.dot(w2[...], h1, preferred_element_type=jnp.float32, precision=prec)
        + b2[...], 0.0)
    u = (jnp.dot(w3[...], h2, preferred_element_type=jnp.float32, precision=prec)
         + b3[...])
    out[0] = u * act


def kernel(x, fire_mask, W1, b1, W2, b2, W3, b3):
    B, C, H, W = x.shape
    nh = H // _HB
    n = _HB * W
    k = _HB // 2
    nhb = H * W // (2 * W)
    xflat = x.reshape(B, C, H * W)
    fmflat = fire_mask.reshape(B, 1, H * W)
    wcol = jnp.arange(n, dtype=jnp.int32) % W
    mle = (wcol != W - 1).astype(jnp.float32).reshape(1, n)
    mre = (wcol != 0).astype(jnp.float32).reshape(1, n)
    b1c = b1.reshape(_EMB, 1)
    b2c = b2.reshape(_EMB, 1)
    b3c = b3.reshape(_CH, 1)

    outf = pl.pallas_call(
        _fused_kernel,
        grid=(B, nh),
        in_specs=[
            pl.BlockSpec((1, C, 2 * W),
                         lambda b, h: (b, 0, jnp.maximum(k * h - 1, 0))),
            pl.BlockSpec((1, C, n), lambda b, h: (b, 0, h)),
            pl.BlockSpec((1, C, 2 * W),
                         lambda b, h: (b, 0, jnp.minimum(k * (h + 1), nhb - 1))),
            pl.BlockSpec((1, 1, n), lambda b, h: (b, 0, h)),
            pl.BlockSpec((1, n), lambda b, h: (0, 0)),
            pl.BlockSpec((1, n), lambda b, h: (0, 0)),
            pl.BlockSpec((_EMB, 3 * _CH), lambda b, h: (0, 0)),
            pl.BlockSpec((_EMB, 1), lambda b, h: (0, 0)),
            pl.BlockSpec((_EMB, _EMB), lambda b, h: (0, 0)),
            pl.BlockSpec((_EMB, 1), lambda b, h: (0, 0)),
            pl.BlockSpec((_CH, _EMB), lambda b, h: (0, 0)),
            pl.BlockSpec((_CH, 1), lambda b, h: (0, 0)),
        ],
        out_specs=pl.BlockSpec((1, C, n), lambda b, h: (b, 0, h)),
        out_shape=jax.ShapeDtypeStruct((B, C, H * W), jnp.float32),
        compiler_params=pltpu.CompilerParams(
            dimension_semantics=("parallel", "arbitrary")),
    )(xflat, xflat, xflat, fmflat, mle, mre, W1, b1c, W2, b2c, W3, b3c)
    return outf.reshape(B, C, H, W)


# speedup vs baseline: 1.0999x; 1.0999x over previous
import jax
import jax.numpy as jnp
from jax.experimental import pallas as pl
from jax.experimental.pallas import tpu as pltpu

_CH = 16
_EMB = 128
_HB = 16
_W = 384


def _fused_kernel(xt, xc, xb, fm, mle, mre, w1, b1, w2, b2, w3, b3, out):
    w = _W
    n = _HB * w
    nh = pl.num_programs(1)
    i = pl.program_id(1)
    top = xt[0] * jnp.where(i > 0, 1.0, 0.0)
    bot = xb[0] * jnp.where(i < nh - 1, 1.0, 0.0)
    xe = jnp.concatenate([top, xc[0], bot], axis=1)

    up = xe[:, w:w + n]
    mid = xe[:, 2 * w:2 * w + n]
    dn = xe[:, 3 * w:3 * w + n]
    ul = xe[:, w + 1:w + 1 + n]
    ur = xe[:, w - 1:w - 1 + n]
    cl = xe[:, 2 * w + 1:2 * w + 1 + n]
    cr = xe[:, 2 * w - 1:2 * w - 1 + n]
    dl = xe[:, 3 * w + 1:3 * w + 1 + n]
    dr = xe[:, 3 * w - 1:3 * w - 1 + n]
    ml = mle[...]
    mr = mre[...]

    gx = (ul + 2.0 * cl + dl) * ml - (ur + 2.0 * cr + dr) * mr
    gy = (dl - ul) * ml + (dr - ur) * mr + 2.0 * (dn - up)

    xa = xe[3:4, :]
    pmf = jnp.maximum(jnp.maximum(xa[:, :n + 2 * w], xa[:, w:n + 3 * w]),
                      xa[:, 2 * w:])
    pooled = jnp.maximum(
        jnp.maximum(pmf[:, w + 1:w + 1 + n] * ml, pmf[:, w - 1:w - 1 + n] * mr),
        pmf[:, w:w + n])
    act = jnp.where((pooled > 0.1) & (fm[0] != 0), 1.0, 0.0)

    f = jnp.concatenate([mid, gx, gy], axis=0)
    prec = jax.lax.Precision.DEFAULT
    h1 = jnp.maximum(
        jnp.dot(w1[...], f, preferred_element_type=jnp.float32, precision=prec)
        + b1[...], 0.0)
    h2 = jnp.maximum(
        jnp.dot(w2[...], h1, preferred_element_type=jnp.float32, precision=prec)
        + b2[...], 0.0)
    u = (jnp.dot(w3[...], h2, preferred_element_type=jnp.float32, precision=prec)
         + b3[...])
    out[0] = u * act


def kernel(x, fire_mask, W1, b1, W2, b2, W3, b3):
    B, C, H, W = x.shape
    nh = H // _HB
    n = _HB * W
    k = _HB // 2
    nhb = H * W // (2 * W)
    xflat = x.reshape(B, C, H * W)
    fmflat = fire_mask.reshape(B, 1, H * W)
    wcol = jnp.arange(n, dtype=jnp.int32) % W
    mle = (wcol != W - 1).astype(jnp.float32).reshape(1, n)
    mre = (wcol != 0).astype(jnp.float32).reshape(1, n)
    b1c = b1.reshape(_EMB, 1)
    b2c = b2.reshape(_EMB, 1)
    b3c = b3.reshape(_CH, 1)

    outf = pl.pallas_call(
        _fused_kernel,
        grid=(B, nh),
        in_specs=[
            pl.BlockSpec((1, C, 2 * W),
                         lambda b, h: (b, 0, jnp.maximum(k * h - 1, 0))),
            pl.BlockSpec((1, C, n), lambda b, h: (b, 0, h)),
            pl.BlockSpec((1, C, 2 * W),
                         lambda b, h: (b, 0, jnp.minimum(k * (h + 1), nhb - 1))),
            pl.BlockSpec((1, 1, n), lambda b, h: (b, 0, h)),
            pl.BlockSpec((1, n), lambda b, h: (0, 0)),
            pl.BlockSpec((1, n), lambda b, h: (0, 0)),
            pl.BlockSpec((_EMB, 3 * _CH), lambda b, h: (0, 0)),
            pl.BlockSpec((_EMB, 1), lambda b, h: (0, 0)),
            pl.BlockSpec((_EMB, _EMB), lambda b, h: (0, 0)),
            pl.BlockSpec((_EMB, 1), lambda b, h: (0, 0)),
            pl.BlockSpec((_CH, _EMB), lambda b, h: (0, 0)),
            pl.BlockSpec((_CH, 1), lambda b, h: (0, 0)),
        ],
        out_specs=pl.BlockSpec((1, C, n), lambda b, h: (b, 0, h)),
        out_shape=jax.ShapeDtypeStruct((B, C, H * W), jnp.float32),
        compiler_params=pltpu.CompilerParams(
            dimension_semantics=("parallel", "arbitrary")),
    )(xflat, xflat, xflat, fmflat, mle, mre, W1, b1c, W2, b2c, W3, b3c)
    return outf.reshape(B, C, H, W)


# speedup vs baseline: 1.5758x; 1.4327x over previous
import jax
import jax.numpy as jnp
from jax.experimental import pallas as pl
from jax.experimental.pallas import tpu as pltpu

_CH = 16
_EMB = 128
_HB = 16
_W = 384


def _fused_kernel(xt, xc, xb, fm, mle, mre, w1, b1, w2, b2, w3, b3, out):
    w = _W
    n = _HB * w
    nh = pl.num_programs(1)
    i = pl.program_id(1)
    top = xt[0][:, 6:8, :].reshape(_CH, 2 * w) * jnp.where(i > 0, 1.0, 0.0)
    bot = xb[0][:, 0:2, :].reshape(_CH, 2 * w) * jnp.where(i < nh - 1, 1.0, 0.0)
    xe = jnp.concatenate([top, xc[0].reshape(_CH, n), bot], axis=1)

    up = xe[:, w:w + n]
    mid = xe[:, 2 * w:2 * w + n]
    dn = xe[:, 3 * w:3 * w + n]
    ul = xe[:, w + 1:w + 1 + n]
    ur = xe[:, w - 1:w - 1 + n]
    cl = xe[:, 2 * w + 1:2 * w + 1 + n]
    cr = xe[:, 2 * w - 1:2 * w - 1 + n]
    dl = xe[:, 3 * w + 1:3 * w + 1 + n]
    dr = xe[:, 3 * w - 1:3 * w - 1 + n]
    ml = mle[...]
    mr = mre[...]

    gx = (ul + 2.0 * cl + dl) * ml - (ur + 2.0 * cr + dr) * mr
    gy = (dl - ul) * ml + (dr - ur) * mr + 2.0 * (dn - up)

    xa = xe[3:4, :]
    pmf = jnp.maximum(jnp.maximum(xa[:, :n + 2 * w], xa[:, w:n + 3 * w]),
                      xa[:, 2 * w:])
    pooled = jnp.maximum(
        jnp.maximum(pmf[:, w + 1:w + 1 + n] * ml, pmf[:, w - 1:w - 1 + n] * mr),
        pmf[:, w:w + n])
    act = jnp.where((pooled > 0.1) & (fm[0].reshape(1, n) != 0), 1.0, 0.0)

    f = jnp.concatenate([mid, gx, gy], axis=0)
    prec = jax.lax.Precision.DEFAULT
    h1 = jnp.maximum(
        jnp.dot(w1[...], f, preferred_element_type=jnp.float32, precision=prec)
        + b1[...], 0.0)
    h2 = jnp.maximum(
        jnp.dot(w2[...], h1, preferred_element_type=jnp.float32, precision=prec)
        + b2[...], 0.0)
    u = (jnp.dot(w3[...], h2, preferred_element_type=jnp.float32, precision=prec)
         + b3[...])
    out[0] = (u * act).reshape(_CH, _HB, w)


def kernel(x, fire_mask, W1, b1, W2, b2, W3, b3):
    B, C, H, W = x.shape
    nh = H // _HB
    n = _HB * W
    nhb = H // 8
    wcol = jnp.arange(n, dtype=jnp.int32) % W
    mle = (wcol != W - 1).astype(jnp.float32).reshape(1, n)
    mre = (wcol != 0).astype(jnp.float32).reshape(1, n)
    b1c = b1.reshape(_EMB, 1)
    b2c = b2.reshape(_EMB, 1)
    b3c = b3.reshape(_CH, 1)
    k = _HB // 8

    return pl.pallas_call(
        _fused_kernel,
        grid=(B, nh),
        in_specs=[
            pl.BlockSpec((1, C, 8, W),
                         lambda b, h: (b, 0, jnp.maximum(k * h - 1, 0), 0)),
            pl.BlockSpec((1, C, _HB, W), lambda b, h: (b, 0, h, 0)),
            pl.BlockSpec((1, C, 8, W),
                         lambda b, h: (b, 0, jnp.minimum(k * (h + 1), nhb - 1), 0)),
            pl.BlockSpec((1, 1, _HB, W), lambda b, h: (b, 0, h, 0)),
            pl.BlockSpec((1, n), lambda b, h: (0, 0)),
            pl.BlockSpec((1, n), lambda b, h: (0, 0)),
            pl.BlockSpec((_EMB, 3 * _CH), lambda b, h: (0, 0)),
            pl.BlockSpec((_EMB, 1), lambda b, h: (0, 0)),
            pl.BlockSpec((_EMB, _EMB), lambda b, h: (0, 0)),
            pl.BlockSpec((_EMB, 1), lambda b, h: (0, 0)),
            pl.BlockSpec((_CH, _EMB), lambda b, h: (0, 0)),
            pl.BlockSpec((_CH, 1), lambda b, h: (0, 0)),
        ],
        out_specs=pl.BlockSpec((1, C, _HB, W), lambda b, h: (b, 0, h, 0)),
        out_shape=jax.ShapeDtypeStruct((B, C, H, W), jnp.float32),
        compiler_params=pltpu.CompilerParams(
            dimension_semantics=("parallel", "arbitrary")),
    )(x, x, x, fire_mask, mle, mre, W1, b1c, W2, b2c, W3, b3c)


# speedup vs baseline: 1.6150x; 1.0249x over previous
import jax
import jax.numpy as jnp
from jax.experimental import pallas as pl
from jax.experimental.pallas import tpu as pltpu

_CH = 16
_EMB = 128
_HB = 16
_W = 384


def _fused_kernel(xt, xc, xb, fm, mle, mre, w1, b1, w2, b2, w3, b3, out):
    w = _W
    n = _HB * w
    nh = pl.num_programs(1)
    i = pl.program_id(1)
    top = xt[0][:, 6:8, :].reshape(_CH, 2 * w) * jnp.where(i > 0, 1.0, 0.0)
    bot = xb[0][:, 0:2, :].reshape(_CH, 2 * w) * jnp.where(i < nh - 1, 1.0, 0.0)
    xe = jnp.concatenate([top, xc[0].reshape(_CH, n), bot], axis=1)

    ml = mle[...]
    mr = mre[...]

    xu = xe[:, :n + 2 * w]
    xm = xe[:, w:n + 3 * w]
    xd = xe[:, 2 * w:]
    s = xu + 2.0 * xm + xd
    t = xd - xu
    gx = s[:, w + 1:w + 1 + n] * ml - s[:, w - 1:w - 1 + n] * mr
    gy = (t[:, w + 1:w + 1 + n] * ml + t[:, w - 1:w - 1 + n] * mr
          + 2.0 * t[:, w:w + n])

    xa = xe[3:4, :]
    pmf = jnp.maximum(jnp.maximum(xa[:, :n + 2 * w], xa[:, w:n + 3 * w]),
                      xa[:, 2 * w:])
    pooled = jnp.maximum(
        jnp.maximum(pmf[:, w + 1:w + 1 + n] * ml, pmf[:, w - 1:w - 1 + n] * mr),
        pmf[:, w:w + n])
    act = jnp.where((pooled > 0.1) & (fm[0].reshape(1, n) != 0), 1.0, 0.0)

    mid = xe[:, 2 * w:2 * w + n]
    f = jnp.concatenate([mid, gx, gy], axis=0).astype(jnp.bfloat16)
    h1 = jnp.dot(w1[...], f, preferred_element_type=jnp.float32)
    h1 = jnp.maximum(h1.astype(jnp.bfloat16) + b1[...], 0)
    h2 = jnp.dot(w2[...], h1, preferred_element_type=jnp.float32)
    h2 = jnp.maximum(h2.astype(jnp.bfloat16) + b2[...], 0)
    u = jnp.dot(w3[...], h2, preferred_element_type=jnp.float32) + b3[...]
    out[0] = (u * act).reshape(_CH, _HB, w)


def kernel(x, fire_mask, W1, b1, W2, b2, W3, b3):
    B, C, H, W = x.shape
    nh = H // _HB
    n = _HB * W
    nhb = H // 8
    wcol = jnp.arange(n, dtype=jnp.int32) % W
    mle = (wcol != W - 1).astype(jnp.float32).reshape(1, n)
    mre = (wcol != 0).astype(jnp.float32).reshape(1, n)
    w1b = W1.astype(jnp.bfloat16)
    w2b = W2.astype(jnp.bfloat16)
    w3b = W3.astype(jnp.bfloat16)
    b1c = b1.astype(jnp.bfloat16).reshape(_EMB, 1)
    b2c = b2.astype(jnp.bfloat16).reshape(_EMB, 1)
    b3c = b3.reshape(_CH, 1)
    k = _HB // 8

    return pl.pallas_call(
        _fused_kernel,
        grid=(B, nh),
        in_specs=[
            pl.BlockSpec((1, C, 8, W),
                         lambda b, h: (b, 0, jnp.maximum(k * h - 1, 0), 0)),
            pl.BlockSpec((1, C, _HB, W), lambda b, h: (b, 0, h, 0)),
            pl.BlockSpec((1, C, 8, W),
                         lambda b, h: (b, 0, jnp.minimum(k * (h + 1), nhb - 1), 0)),
            pl.BlockSpec((1, 1, _HB, W), lambda b, h: (b, 0, h, 0)),
            pl.BlockSpec((1, n), lambda b, h: (0, 0)),
            pl.BlockSpec((1, n), lambda b, h: (0, 0)),
            pl.BlockSpec((_EMB, 3 * _CH), lambda b, h: (0, 0)),
            pl.BlockSpec((_EMB, 1), lambda b, h: (0, 0)),
            pl.BlockSpec((_EMB, _EMB), lambda b, h: (0, 0)),
            pl.BlockSpec((_EMB, 1), lambda b, h: (0, 0)),
            pl.BlockSpec((_CH, _EMB), lambda b, h: (0, 0)),
            pl.BlockSpec((_CH, 1), lambda b, h: (0, 0)),
        ],
        out_specs=pl.BlockSpec((1, C, _HB, W), lambda b, h: (b, 0, h, 0)),
        out_shape=jax.ShapeDtypeStruct((B, C, H, W), jnp.float32),
        compiler_params=pltpu.CompilerParams(
            dimension_semantics=("parallel", "arbitrary")),
    )(x, x, x, fire_mask, mle, mre, w1b, b1c, w2b, b2c, w3b, b3c)


# speedup vs baseline: 1.6230x; 1.0050x over previous
import jax
import jax.numpy as jnp
from jax.experimental import pallas as pl
from jax.experimental.pallas import tpu as pltpu

_CH = 16
_EMB = 128
_HB = 16
_W = 384


def _fused_kernel(xt, xc, xb, fm, mle, mre, mleb, mreb,
                  w1, b1, w2, b2, w3, b3, out):
    w = _W
    n = _HB * w
    nh = pl.num_programs(1)
    i = pl.program_id(1)
    tfac = jnp.where(i > 0, 1.0, 0.0)
    bfac = jnp.where(i < nh - 1, 1.0, 0.0)

    topb = (xt[0][:, 6:8, :].astype(jnp.bfloat16).reshape(_CH, 2 * w)
            * tfac.astype(jnp.bfloat16))
    botb = (xb[0][:, 0:2, :].astype(jnp.bfloat16).reshape(_CH, 2 * w)
            * bfac.astype(jnp.bfloat16))
    xeb = jnp.concatenate(
        [topb, xc[0].astype(jnp.bfloat16).reshape(_CH, n), botb], axis=1)

    mlb = mleb[...]
    mrb = mreb[...]

    xu = xeb[:, :n + 2 * w]
    xm = xeb[:, w:n + 3 * w]
    xd = xeb[:, 2 * w:]
    s = xu + 2.0 * xm + xd
    t = xd - xu
    gx = s[:, w + 1:w + 1 + n] * mlb - s[:, w - 1:w - 1 + n] * mrb
    gy = (t[:, w + 1:w + 1 + n] * mlb + t[:, w - 1:w - 1 + n] * mrb
          + 2.0 * t[:, w:w + n])
    mid = xeb[:, 2 * w:2 * w + n]
    f = jnp.concatenate([mid, gx, gy], axis=0)

    ta = xt[0][3:4, 6:8, :].reshape(1, 2 * w) * tfac
    ba = xb[0][3:4, 0:2, :].reshape(1, 2 * w) * bfac
    ae = jnp.concatenate([ta, xc[0][3:4].reshape(1, n), ba], axis=1)
    pmf = jnp.maximum(jnp.maximum(ae[:, :n + 2 * w], ae[:, w:n + 3 * w]),
                      ae[:, 2 * w:])
    ml = mle[...]
    mr = mre[...]
    pooled = jnp.maximum(
        jnp.maximum(pmf[:, w + 1:w + 1 + n] * ml, pmf[:, w - 1:w - 1 + n] * mr),
        pmf[:, w:w + n])
    act = jnp.where((pooled > 0.1) & (fm[0].reshape(1, n) != 0), 1.0, 0.0)

    h1 = jnp.dot(w1[...], f, preferred_element_type=jnp.float32)
    h1 = jnp.maximum(h1.astype(jnp.bfloat16) + b1[...], 0)
    h2 = jnp.dot(w2[...], h1, preferred_element_type=jnp.float32)
    h2 = jnp.maximum(h2.astype(jnp.bfloat16) + b2[...], 0)
    u = jnp.dot(w3[...], h2, preferred_element_type=jnp.float32) + b3[...]
    out[0] = (u * act).reshape(_CH, _HB, w)


def kernel(x, fire_mask, W1, b1, W2, b2, W3, b3):
    B, C, H, W = x.shape
    nh = H // _HB
    n = _HB * W
    nhb = H // 8
    wcol = jnp.arange(n, dtype=jnp.int32) % W
    mle = (wcol != W - 1).astype(jnp.float32).reshape(1, n)
    mre = (wcol != 0).astype(jnp.float32).reshape(1, n)
    w1b = W1.astype(jnp.bfloat16)
    w2b = W2.astype(jnp.bfloat16)
    w3b = W3.astype(jnp.bfloat16)
    b1c = b1.astype(jnp.bfloat16).reshape(_EMB, 1)
    b2c = b2.astype(jnp.bfloat16).reshape(_EMB, 1)
    b3c = b3.reshape(_CH, 1)
    k = _HB // 8

    def const_spec(shape):
        return pl.BlockSpec(shape, lambda b, h: tuple(0 for _ in shape))

    return pl.pallas_call(
        _fused_kernel,
        grid=(B, nh),
        in_specs=[
            pl.BlockSpec((1, C, 8, W),
                         lambda b, h: (b, 0, jnp.maximum(k * h - 1, 0), 0)),
            pl.BlockSpec((1, C, _HB, W), lambda b, h: (b, 0, h, 0)),
            pl.BlockSpec((1, C, 8, W),
                         lambda b, h: (b, 0, jnp.minimum(k * (h + 1), nhb - 1), 0)),
            pl.BlockSpec((1, 1, _HB, W), lambda b, h: (b, 0, h, 0)),
            const_spec((1, n)),
            const_spec((1, n)),
            const_spec((1, n)),
            const_spec((1, n)),
            const_spec((_EMB, 3 * _CH)),
            const_spec((_EMB, 1)),
            const_spec((_EMB, _EMB)),
            const_spec((_EMB, 1)),
            const_spec((_CH, _EMB)),
            const_spec((_CH, 1)),
        ],
        out_specs=pl.BlockSpec((1, C, _HB, W), lambda b, h: (b, 0, h, 0)),
        out_shape=jax.ShapeDtypeStruct((B, C, H, W), jnp.float32),
        compiler_params=pltpu.CompilerParams(
            dimension_semantics=("parallel", "arbitrary")),
    )(x, x, x, fire_mask, mle, mre, mle.astype(jnp.bfloat16),
      mre.astype(jnp.bfloat16), w1b, b1c, w2b, b2c, w3b, b3c)


# speedup vs baseline: 1.7369x; 1.0701x over previous
import jax
import jax.numpy as jnp
from jax.experimental import pallas as pl
from jax.experimental.pallas import tpu as pltpu

_CH = 16
_EMB = 128
_HB = 16
_W = 384


def _fused_kernel(xt, xc, xb, fm, mle, mre, mleb, mreb,
                  w1, b1, w2, b2, w3, b3, out):
    w = _W
    n = _HB * w
    nh = pl.num_programs(1)
    i = pl.program_id(1)
    tfac = jnp.where(i > 0, 1.0, 0.0)
    bfac = jnp.where(i < nh - 1, 1.0, 0.0)

    topb = (xt[0][:, 6:8, :].astype(jnp.bfloat16).reshape(_CH, 2 * w)
            * tfac.astype(jnp.bfloat16))
    botb = (xb[0][:, 0:2, :].astype(jnp.bfloat16).reshape(_CH, 2 * w)
            * bfac.astype(jnp.bfloat16))
    xeb = jnp.concatenate(
        [topb, xc[0].astype(jnp.bfloat16).reshape(_CH, n), botb], axis=1)

    mlb = mleb[...]
    mrb = mreb[...]

    xu = xeb[:, :n + 2 * w]
    xm = xeb[:, w:n + 3 * w]
    xd = xeb[:, 2 * w:]
    s = xu + 2.0 * xm + xd
    t = xd - xu

    ta = xt[0][3:4, 6:8, :].reshape(1, 2 * w) * tfac
    ba = xb[0][3:4, 0:2, :].reshape(1, 2 * w) * bfac
    ae = jnp.concatenate([ta, xc[0][3:4].reshape(1, n), ba], axis=1)
    pmf = jnp.maximum(jnp.maximum(ae[:, :n + 2 * w], ae[:, w:n + 3 * w]),
                      ae[:, 2 * w:])
    ml = mle[...]
    mr = mre[...]
    pooled = jnp.maximum(
        jnp.maximum(pmf[:, w + 1:w + 1 + n] * ml, pmf[:, w - 1:w - 1 + n] * mr),
        pmf[:, w:w + n])
    act = jnp.where((pooled > 0.1) & (fm[0].reshape(1, n) != 0), 1.0, 0.0)

    us = []
    n2 = n // 2
    for half in range(2):
        o = half * n2
        gx = (s[:, w + 1 + o:w + 1 + o + n2] * mlb[:, o:o + n2]
              - s[:, w - 1 + o:w - 1 + o + n2] * mrb[:, o:o + n2])
        gy = (t[:, w + 1 + o:w + 1 + o + n2] * mlb[:, o:o + n2]
              + t[:, w - 1 + o:w - 1 + o + n2] * mrb[:, o:o + n2]
              + 2.0 * t[:, w + o:w + o + n2])
        mid = xeb[:, 2 * w + o:2 * w + o + n2]
        f = jnp.concatenate([mid, gx, gy], axis=0)
        h1 = jnp.dot(w1[...], f, preferred_element_type=jnp.float32)
        h1 = jnp.maximum(h1.astype(jnp.bfloat16) + b1[...], 0)
        h2 = jnp.dot(w2[...], h1, preferred_element_type=jnp.float32)
        h2 = jnp.maximum(h2.astype(jnp.bfloat16) + b2[...], 0)
        us.append(jnp.dot(w3[...], h2, preferred_element_type=jnp.float32))
    u = jnp.concatenate(us, axis=1) + b3[...]
    out[0] = (u * act).reshape(_CH, _HB, w)


def kernel(x, fire_mask, W1, b1, W2, b2, W3, b3):
    B, C, H, W = x.shape
    nh = H // _HB
    n = _HB * W
    nhb = H // 8
    wcol = jnp.arange(n, dtype=jnp.int32) % W
    mle = (wcol != W - 1).astype(jnp.float32).reshape(1, n)
    mre = (wcol != 0).astype(jnp.float32).reshape(1, n)
    w1b = W1.astype(jnp.bfloat16)
    w2b = W2.astype(jnp.bfloat16)
    w3b = W3.astype(jnp.bfloat16)
    b1c = b1.astype(jnp.bfloat16).reshape(_EMB, 1)
    b2c = b2.astype(jnp.bfloat16).reshape(_EMB, 1)
    b3c = b3.reshape(_CH, 1)
    k = _HB // 8

    def const_spec(shape):
        return pl.BlockSpec(shape, lambda b, h: tuple(0 for _ in shape))

    return pl.pallas_call(
        _fused_kernel,
        grid=(B, nh),
        in_specs=[
            pl.BlockSpec((1, C, 8, W),
                         lambda b, h: (b, 0, jnp.maximum(k * h - 1, 0), 0)),
            pl.BlockSpec((1, C, _HB, W), lambda b, h: (b, 0, h, 0)),
            pl.BlockSpec((1, C, 8, W),
                         lambda b, h: (b, 0, jnp.minimum(k * (h + 1), nhb - 1), 0)),
            pl.BlockSpec((1, 1, _HB, W), lambda b, h: (b, 0, h, 0)),
            const_spec((1, n)),
            const_spec((1, n)),
            const_spec((1, n)),
            const_spec((1, n)),
            const_spec((_EMB, 3 * _CH)),
            const_spec((_EMB, 1)),
            const_spec((_EMB, _EMB)),
            const_spec((_EMB, 1)),
            const_spec((_CH, _EMB)),
            const_spec((_CH, 1)),
        ],
        out_specs=pl.BlockSpec((1, C, _HB, W), lambda b, h: (b, 0, h, 0)),
        out_shape=jax.ShapeDtypeStruct((B, C, H, W), jnp.float32),
        compiler_params=pltpu.CompilerParams(
            dimension_semantics=("parallel", "arbitrary")),
    )(x, x, x, fire_mask, mle, mre, mle.astype(jnp.bfloat16),
      mre.astype(jnp.bfloat16), w1b, b1c, w2b, b2c, w3b, b3c)


# speedup vs baseline: 1.9101x; 1.0997x over previous
import jax
import jax.numpy as jnp
from jax.experimental import pallas as pl
from jax.experimental.pallas import tpu as pltpu

_CH = 16
_EMB = 128
_HB = 32
_W = 384


def _fused_kernel(xt, xc, xb, fm, mle, mre, mleb, mreb,
                  w1, b1, w2, b2, w3, b3, out):
    w = _W
    n = _HB * w
    nh = pl.num_programs(1)
    i = pl.program_id(1)
    tfac = jnp.where(i > 0, 1.0, 0.0)
    bfac = jnp.where(i < nh - 1, 1.0, 0.0)

    topb = (xt[0][:, 6:8, :].astype(jnp.bfloat16).reshape(_CH, 2 * w)
            * tfac.astype(jnp.bfloat16))
    botb = (xb[0][:, 0:2, :].astype(jnp.bfloat16).reshape(_CH, 2 * w)
            * bfac.astype(jnp.bfloat16))
    xeb = jnp.concatenate(
        [topb, xc[0].astype(jnp.bfloat16).reshape(_CH, n), botb], axis=1)

    mlb = mleb[...]
    mrb = mreb[...]

    xu = xeb[:, :n + 2 * w]
    xm = xeb[:, w:n + 3 * w]
    xd = xeb[:, 2 * w:]
    s = xu + 2.0 * xm + xd
    t = xd - xu

    ta = xt[0][3:4, 6:8, :].reshape(1, 2 * w) * tfac
    ba = xb[0][3:4, 0:2, :].reshape(1, 2 * w) * bfac
    ae = jnp.concatenate([ta, xc[0][3:4].reshape(1, n), ba], axis=1)
    pmf = jnp.maximum(jnp.maximum(ae[:, :n + 2 * w], ae[:, w:n + 3 * w]),
                      ae[:, 2 * w:])
    ml = mle[...]
    mr = mre[...]
    pooled = jnp.maximum(
        jnp.maximum(pmf[:, w + 1:w + 1 + n] * ml, pmf[:, w - 1:w - 1 + n] * mr),
        pmf[:, w:w + n])
    act = jnp.where((pooled > 0.1) & (fm[0].reshape(1, n) != 0), 1.0, 0.0)

    us = []
    n2 = n // 2
    for half in range(2):
        o = half * n2
        gx = (s[:, w + 1 + o:w + 1 + o + n2] * mlb[:, o:o + n2]
              - s[:, w - 1 + o:w - 1 + o + n2] * mrb[:, o:o + n2])
        gy = (t[:, w + 1 + o:w + 1 + o + n2] * mlb[:, o:o + n2]
              + t[:, w - 1 + o:w - 1 + o + n2] * mrb[:, o:o + n2]
              + 2.0 * t[:, w + o:w + o + n2])
        mid = xeb[:, 2 * w + o:2 * w + o + n2]
        f = jnp.concatenate([mid, gx, gy], axis=0)
        h1 = jnp.dot(w1[...], f, preferred_element_type=jnp.float32)
        h1 = jnp.maximum(h1.astype(jnp.bfloat16) + b1[...], 0)
        h2 = jnp.dot(w2[...], h1, preferred_element_type=jnp.float32)
        h2 = jnp.maximum(h2.astype(jnp.bfloat16) + b2[...], 0)
        us.append(jnp.dot(w3[...], h2, preferred_element_type=jnp.float32))
    u = jnp.concatenate(us, axis=1) + b3[...]
    out[0] = (u * act).reshape(_CH, _HB, w)


def kernel(x, fire_mask, W1, b1, W2, b2, W3, b3):
    B, C, H, W = x.shape
    nh = H // _HB
    n = _HB * W
    nhb = H // 8
    wcol = jnp.arange(n, dtype=jnp.int32) % W
    mle = (wcol != W - 1).astype(jnp.float32).reshape(1, n)
    mre = (wcol != 0).astype(jnp.float32).reshape(1, n)
    w1b = W1.astype(jnp.bfloat16)
    w2b = W2.astype(jnp.bfloat16)
    w3b = W3.astype(jnp.bfloat16)
    b1c = b1.astype(jnp.bfloat16).reshape(_EMB, 1)
    b2c = b2.astype(jnp.bfloat16).reshape(_EMB, 1)
    b3c = b3.reshape(_CH, 1)
    k = _HB // 8

    def const_spec(shape):
        return pl.BlockSpec(shape, lambda b, h: tuple(0 for _ in shape))

    return pl.pallas_call(
        _fused_kernel,
        grid=(B, nh),
        in_specs=[
            pl.BlockSpec((1, C, 8, W),
                         lambda b, h: (b, 0, jnp.maximum(k * h - 1, 0), 0)),
            pl.BlockSpec((1, C, _HB, W), lambda b, h: (b, 0, h, 0)),
            pl.BlockSpec((1, C, 8, W),
                         lambda b, h: (b, 0, jnp.minimum(k * (h + 1), nhb - 1), 0)),
            pl.BlockSpec((1, 1, _HB, W), lambda b, h: (b, 0, h, 0)),
            const_spec((1, n)),
            const_spec((1, n)),
            const_spec((1, n)),
            const_spec((1, n)),
            const_spec((_EMB, 3 * _CH)),
            const_spec((_EMB, 1)),
            const_spec((_EMB, _EMB)),
            const_spec((_EMB, 1)),
            const_spec((_CH, _EMB)),
            const_spec((_CH, 1)),
        ],
        out_specs=pl.BlockSpec((1, C, _HB, W), lambda b, h: (b, 0, h, 0)),
        out_shape=jax.ShapeDtypeStruct((B, C, H, W), jnp.float32),
        compiler_params=pltpu.CompilerParams(
            dimension_semantics=("parallel", "arbitrary")),
    )(x, x, x, fire_mask, mle, mre, mle.astype(jnp.bfloat16),
      mre.astype(jnp.bfloat16), w1b, b1c, w2b, b2c, w3b, b3c)


# speedup vs baseline: 2.0865x; 1.0923x over previous
import jax
import jax.numpy as jnp
from jax.experimental import pallas as pl
from jax.experimental.pallas import tpu as pltpu

_CH = 16
_EMB = 128
_HB = 64
_W = 384


def _fused_kernel(xt, xc, xb, fm, mle, mre, mleb, mreb,
                  w1, b1, w2, b2, w3, b3, out):
    w = _W
    n = _HB * w
    nh = pl.num_programs(1)
    i = pl.program_id(1)
    tfac = jnp.where(i > 0, 1.0, 0.0)
    bfac = jnp.where(i < nh - 1, 1.0, 0.0)

    topb = (xt[0][:, 6:8, :].astype(jnp.bfloat16).reshape(_CH, 2 * w)
            * tfac.astype(jnp.bfloat16))
    botb = (xb[0][:, 0:2, :].astype(jnp.bfloat16).reshape(_CH, 2 * w)
            * bfac.astype(jnp.bfloat16))
    xeb = jnp.concatenate(
        [topb, xc[0].astype(jnp.bfloat16).reshape(_CH, n), botb], axis=1)

    mlb = mleb[...]
    mrb = mreb[...]

    xu = xeb[:, :n + 2 * w]
    xm = xeb[:, w:n + 3 * w]
    xd = xeb[:, 2 * w:]
    s = xu + 2.0 * xm + xd
    t = xd - xu

    ta = xt[0][3:4, 6:8, :].reshape(1, 2 * w) * tfac
    ba = xb[0][3:4, 0:2, :].reshape(1, 2 * w) * bfac
    ae = jnp.concatenate([ta, xc[0][3:4].reshape(1, n), ba], axis=1)
    pmf = jnp.maximum(jnp.maximum(ae[:, :n + 2 * w], ae[:, w:n + 3 * w]),
                      ae[:, 2 * w:])
    ml = mle[...]
    mr = mre[...]
    pooled = jnp.maximum(
        jnp.maximum(pmf[:, w + 1:w + 1 + n] * ml, pmf[:, w - 1:w - 1 + n] * mr),
        pmf[:, w:w + n])
    act = jnp.where((pooled > 0.1) & (fm[0].reshape(1, n) != 0), 1.0, 0.0)

    us = []
    n2 = n // 4
    for half in range(4):
        o = half * n2
        gx = (s[:, w + 1 + o:w + 1 + o + n2] * mlb[:, o:o + n2]
              - s[:, w - 1 + o:w - 1 + o + n2] * mrb[:, o:o + n2])
        gy = (t[:, w + 1 + o:w + 1 + o + n2] * mlb[:, o:o + n2]
              + t[:, w - 1 + o:w - 1 + o + n2] * mrb[:, o:o + n2]
              + 2.0 * t[:, w + o:w + o + n2])
        mid = xeb[:, 2 * w + o:2 * w + o + n2]
        f = jnp.concatenate([mid, gx, gy], axis=0)
        h1 = jnp.dot(w1[...], f, preferred_element_type=jnp.float32)
        h1 = jnp.maximum(h1.astype(jnp.bfloat16) + b1[...], 0)
        h2 = jnp.dot(w2[...], h1, preferred_element_type=jnp.float32)
        h2 = jnp.maximum(h2.astype(jnp.bfloat16) + b2[...], 0)
        us.append(jnp.dot(w3[...], h2, preferred_element_type=jnp.float32))
    u = jnp.concatenate(us, axis=1) + b3[...]
    out[0] = (u * act).reshape(_CH, _HB, w)


def kernel(x, fire_mask, W1, b1, W2, b2, W3, b3):
    B, C, H, W = x.shape
    nh = H // _HB
    n = _HB * W
    nhb = H // 8
    wcol = jnp.arange(n, dtype=jnp.int32) % W
    mle = (wcol != W - 1).astype(jnp.float32).reshape(1, n)
    mre = (wcol != 0).astype(jnp.float32).reshape(1, n)
    w1b = W1.astype(jnp.bfloat16)
    w2b = W2.astype(jnp.bfloat16)
    w3b = W3.astype(jnp.bfloat16)
    b1c = b1.astype(jnp.bfloat16).reshape(_EMB, 1)
    b2c = b2.astype(jnp.bfloat16).reshape(_EMB, 1)
    b3c = b3.reshape(_CH, 1)
    k = _HB // 8

    def const_spec(shape):
        return pl.BlockSpec(shape, lambda b, h: tuple(0 for _ in shape))

    return pl.pallas_call(
        _fused_kernel,
        grid=(B, nh),
        in_specs=[
            pl.BlockSpec((1, C, 8, W),
                         lambda b, h: (b, 0, jnp.maximum(k * h - 1, 0), 0)),
            pl.BlockSpec((1, C, _HB, W), lambda b, h: (b, 0, h, 0)),
            pl.BlockSpec((1, C, 8, W),
                         lambda b, h: (b, 0, jnp.minimum(k * (h + 1), nhb - 1), 0)),
            pl.BlockSpec((1, 1, _HB, W), lambda b, h: (b, 0, h, 0)),
            const_spec((1, n)),
            const_spec((1, n)),
            const_spec((1, n)),
            const_spec((1, n)),
            const_spec((_EMB, 3 * _CH)),
            const_spec((_EMB, 1)),
            const_spec((_EMB, _EMB)),
            const_spec((_EMB, 1)),
            const_spec((_CH, _EMB)),
            const_spec((_CH, 1)),
        ],
        out_specs=pl.BlockSpec((1, C, _HB, W), lambda b, h: (b, 0, h, 0)),
        out_shape=jax.ShapeDtypeStruct((B, C, H, W), jnp.float32),
        compiler_params=pltpu.CompilerParams(
            dimension_semantics=("parallel", "arbitrary")),
    )(x, x, x, fire_mask, mle, mre, mle.astype(jnp.bfloat16),
      mre.astype(jnp.bfloat16), w1b, b1c, w2b, b2c, w3b, b3c)


# speedup vs baseline: 2.1886x; 1.0489x over previous
import jax
import jax.numpy as jnp
from jax.experimental import pallas as pl
from jax.experimental.pallas import tpu as pltpu

_CH = 16
_EMB = 128
_HB = 128
_W = 384


def _fused_kernel(xt, xc, xb, fm, mle, mre, mleb, mreb,
                  w1, b1, w2, b2, w3, b3, out):
    w = _W
    n = _HB * w
    nh = pl.num_programs(1)
    i = pl.program_id(1)
    tfac = jnp.where(i > 0, 1.0, 0.0)
    bfac = jnp.where(i < nh - 1, 1.0, 0.0)

    topb = (xt[0][:, 6:8, :].astype(jnp.bfloat16).reshape(_CH, 2 * w)
            * tfac.astype(jnp.bfloat16))
    botb = (xb[0][:, 0:2, :].astype(jnp.bfloat16).reshape(_CH, 2 * w)
            * bfac.astype(jnp.bfloat16))
    xeb = jnp.concatenate(
        [topb, xc[0].astype(jnp.bfloat16).reshape(_CH, n), botb], axis=1)

    mlb = mleb[...]
    mrb = mreb[...]

    xu = xeb[:, :n + 2 * w]
    xm = xeb[:, w:n + 3 * w]
    xd = xeb[:, 2 * w:]
    s = xu + 2.0 * xm + xd
    t = xd - xu

    ta = xt[0][3:4, 6:8, :].reshape(1, 2 * w) * tfac
    ba = xb[0][3:4, 0:2, :].reshape(1, 2 * w) * bfac
    ae = jnp.concatenate([ta, xc[0][3:4].reshape(1, n), ba], axis=1)
    pmf = jnp.maximum(jnp.maximum(ae[:, :n + 2 * w], ae[:, w:n + 3 * w]),
                      ae[:, 2 * w:])
    ml = mle[...]
    mr = mre[...]
    pooled = jnp.maximum(
        jnp.maximum(pmf[:, w + 1:w + 1 + n] * ml, pmf[:, w - 1:w - 1 + n] * mr),
        pmf[:, w:w + n])
    act = jnp.where((pooled > 0.1) & (fm[0].reshape(1, n) != 0), 1.0, 0.0)

    us = []
    n2 = n // 8
    for half in range(8):
        o = half * n2
        gx = (s[:, w + 1 + o:w + 1 + o + n2] * mlb[:, o:o + n2]
              - s[:, w - 1 + o:w - 1 + o + n2] * mrb[:, o:o + n2])
        gy = (t[:, w + 1 + o:w + 1 + o + n2] * mlb[:, o:o + n2]
              + t[:, w - 1 + o:w - 1 + o + n2] * mrb[:, o:o + n2]
              + 2.0 * t[:, w + o:w + o + n2])
        mid = xeb[:, 2 * w + o:2 * w + o + n2]
        f = jnp.concatenate([mid, gx, gy], axis=0)
        h1 = jnp.dot(w1[...], f, preferred_element_type=jnp.float32)
        h1 = jnp.maximum(h1.astype(jnp.bfloat16) + b1[...], 0)
        h2 = jnp.dot(w2[...], h1, preferred_element_type=jnp.float32)
        h2 = jnp.maximum(h2.astype(jnp.bfloat16) + b2[...], 0)
        us.append(jnp.dot(w3[...], h2, preferred_element_type=jnp.float32))
    u = jnp.concatenate(us, axis=1) + b3[...]
    out[0] = (u * act).reshape(_CH, _HB, w)


def kernel(x, fire_mask, W1, b1, W2, b2, W3, b3):
    B, C, H, W = x.shape
    nh = H // _HB
    n = _HB * W
    nhb = H // 8
    wcol = jnp.arange(n, dtype=jnp.int32) % W
    mle = (wcol != W - 1).astype(jnp.float32).reshape(1, n)
    mre = (wcol != 0).astype(jnp.float32).reshape(1, n)
    w1b = W1.astype(jnp.bfloat16)
    w2b = W2.astype(jnp.bfloat16)
    w3b = W3.astype(jnp.bfloat16)
    b1c = b1.astype(jnp.bfloat16).reshape(_EMB, 1)
    b2c = b2.astype(jnp.bfloat16).reshape(_EMB, 1)
    b3c = b3.reshape(_CH, 1)
    k = _HB // 8

    def const_spec(shape):
        return pl.BlockSpec(shape, lambda b, h: tuple(0 for _ in shape))

    return pl.pallas_call(
        _fused_kernel,
        grid=(B, nh),
        in_specs=[
            pl.BlockSpec((1, C, 8, W),
                         lambda b, h: (b, 0, jnp.maximum(k * h - 1, 0), 0)),
            pl.BlockSpec((1, C, _HB, W), lambda b, h: (b, 0, h, 0)),
            pl.BlockSpec((1, C, 8, W),
                         lambda b, h: (b, 0, jnp.minimum(k * (h + 1), nhb - 1), 0)),
            pl.BlockSpec((1, 1, _HB, W), lambda b, h: (b, 0, h, 0)),
            const_spec((1, n)),
            const_spec((1, n)),
            const_spec((1, n)),
            const_spec((1, n)),
            const_spec((_EMB, 3 * _CH)),
            const_spec((_EMB, 1)),
            const_spec((_EMB, _EMB)),
            const_spec((_EMB, 1)),
            const_spec((_CH, _EMB)),
            const_spec((_CH, 1)),
        ],
        out_specs=pl.BlockSpec((1, C, _HB, W), lambda b, h: (b, 0, h, 0)),
        out_shape=jax.ShapeDtypeStruct((B, C, H, W), jnp.float32),
        compiler_params=pltpu.CompilerParams(
            dimension_semantics=("parallel", "arbitrary")),
    )(x, x, x, fire_mask, mle, mre, mle.astype(jnp.bfloat16),
      mre.astype(jnp.bfloat16), w1b, b1c, w2b, b2c, w3b, b3c)


# speedup vs baseline: 2.1993x; 1.0049x over previous
import jax
import jax.numpy as jnp
from jax.experimental import pallas as pl
from jax.experimental.pallas import tpu as pltpu

_CH = 16
_EMB = 128
_HB = 192
_W = 384


def _fused_kernel(xt, xc, xb, fm, mle, mre, mleb, mreb,
                  w1, b1, w2, b2, w3, b3, out):
    w = _W
    n = _HB * w
    nh = pl.num_programs(1)
    i = pl.program_id(1)
    tfac = jnp.where(i > 0, 1.0, 0.0)
    bfac = jnp.where(i < nh - 1, 1.0, 0.0)

    topb = (xt[0][:, 6:8, :].astype(jnp.bfloat16).reshape(_CH, 2 * w)
            * tfac.astype(jnp.bfloat16))
    botb = (xb[0][:, 0:2, :].astype(jnp.bfloat16).reshape(_CH, 2 * w)
            * bfac.astype(jnp.bfloat16))
    xeb = jnp.concatenate(
        [topb, xc[0].astype(jnp.bfloat16).reshape(_CH, n), botb], axis=1)

    mlb = mleb[...]
    mrb = mreb[...]

    xu = xeb[:, :n + 2 * w]
    xm = xeb[:, w:n + 3 * w]
    xd = xeb[:, 2 * w:]
    s = xu + 2.0 * xm + xd
    t = xd - xu

    ta = xt[0][3:4, 6:8, :].reshape(1, 2 * w) * tfac
    ba = xb[0][3:4, 0:2, :].reshape(1, 2 * w) * bfac
    ae = jnp.concatenate([ta, xc[0][3:4].reshape(1, n), ba], axis=1)
    pmf = jnp.maximum(jnp.maximum(ae[:, :n + 2 * w], ae[:, w:n + 3 * w]),
                      ae[:, 2 * w:])
    ml = mle[...]
    mr = mre[...]
    pooled = jnp.maximum(
        jnp.maximum(pmf[:, w + 1:w + 1 + n] * ml, pmf[:, w - 1:w - 1 + n] * mr),
        pmf[:, w:w + n])
    act = jnp.where((pooled > 0.1) & (fm[0].reshape(1, n) != 0), 1.0, 0.0)

    us = []
    n2 = n // 12
    for half in range(12):
        o = half * n2
        gx = (s[:, w + 1 + o:w + 1 + o + n2] * mlb[:, o:o + n2]
              - s[:, w - 1 + o:w - 1 + o + n2] * mrb[:, o:o + n2])
        gy = (t[:, w + 1 + o:w + 1 + o + n2] * mlb[:, o:o + n2]
              + t[:, w - 1 + o:w - 1 + o + n2] * mrb[:, o:o + n2]
              + 2.0 * t[:, w + o:w + o + n2])
        mid = xeb[:, 2 * w + o:2 * w + o + n2]
        f = jnp.concatenate([mid, gx, gy], axis=0)
        h1 = jnp.dot(w1[...], f, preferred_element_type=jnp.float32)
        h1 = jnp.maximum(h1.astype(jnp.bfloat16) + b1[...], 0)
        h2 = jnp.dot(w2[...], h1, preferred_element_type=jnp.float32)
        h2 = jnp.maximum(h2.astype(jnp.bfloat16) + b2[...], 0)
        us.append(jnp.dot(w3[...], h2, preferred_element_type=jnp.float32))
    u = jnp.concatenate(us, axis=1) + b3[...]
    out[0] = (u * act).reshape(_CH, _HB, w)


def kernel(x, fire_mask, W1, b1, W2, b2, W3, b3):
    B, C, H, W = x.shape
    nh = H // _HB
    n = _HB * W
    nhb = H // 8
    wcol = jnp.arange(n, dtype=jnp.int32) % W
    mle = (wcol != W - 1).astype(jnp.float32).reshape(1, n)
    mre = (wcol != 0).astype(jnp.float32).reshape(1, n)
    w1b = W1.astype(jnp.bfloat16)
    w2b = W2.astype(jnp.bfloat16)
    w3b = W3.astype(jnp.bfloat16)
    b1c = b1.astype(jnp.bfloat16).reshape(_EMB, 1)
    b2c = b2.astype(jnp.bfloat16).reshape(_EMB, 1)
    b3c = b3.reshape(_CH, 1)
    k = _HB // 8

    def const_spec(shape):
        return pl.BlockSpec(shape, lambda b, h: tuple(0 for _ in shape))

    return pl.pallas_call(
        _fused_kernel,
        grid=(B, nh),
        in_specs=[
            pl.BlockSpec((1, C, 8, W),
                         lambda b, h: (b, 0, jnp.maximum(k * h - 1, 0), 0)),
            pl.BlockSpec((1, C, _HB, W), lambda b, h: (b, 0, h, 0)),
            pl.BlockSpec((1, C, 8, W),
                         lambda b, h: (b, 0, jnp.minimum(k * (h + 1), nhb - 1), 0)),
            pl.BlockSpec((1, 1, _HB, W), lambda b, h: (b, 0, h, 0)),
            const_spec((1, n)),
            const_spec((1, n)),
            const_spec((1, n)),
            const_spec((1, n)),
            const_spec((_EMB, 3 * _CH)),
            const_spec((_EMB, 1)),
            const_spec((_EMB, _EMB)),
            const_spec((_EMB, 1)),
            const_spec((_CH, _EMB)),
            const_spec((_CH, 1)),
        ],
        out_specs=pl.BlockSpec((1, C, _HB, W), lambda b, h: (b, 0, h, 0)),
        out_shape=jax.ShapeDtypeStruct((B, C, H, W), jnp.float32),
        compiler_params=pltpu.CompilerParams(
            dimension_semantics=("parallel", "arbitrary")),
    )(x, x, x, fire_mask, mle, mre, mle.astype(jnp.bfloat16),
      mre.astype(jnp.bfloat16), w1b, b1c, w2b, b2c, w3b, b3c)


# speedup vs baseline: 2.2039x; 1.0021x over previous
import jax
import jax.numpy as jnp
from jax.experimental import pallas as pl
from jax.experimental.pallas import tpu as pltpu

_CH = 16
_EMB = 128
_HB = 192
_W = 384


def _fused_kernel(xt, xc, xb, fm, mle, mre, mleb, mreb,
                  w1, b1, w2, b2, w3, b3, out):
    w = _W
    n = _HB * w
    nh = pl.num_programs(1)
    i = pl.program_id(1)
    tfac = jnp.where(i > 0, 1.0, 0.0)
    bfac = jnp.where(i < nh - 1, 1.0, 0.0)

    topb = (xt[0][:, 6:8, :].astype(jnp.bfloat16).reshape(_CH, 2 * w)
            * tfac.astype(jnp.bfloat16))
    botb = (xb[0][:, 0:2, :].astype(jnp.bfloat16).reshape(_CH, 2 * w)
            * bfac.astype(jnp.bfloat16))
    xeb = jnp.concatenate(
        [topb, xc[0].astype(jnp.bfloat16).reshape(_CH, n), botb], axis=1)

    mlb = mleb[...]
    mrb = mreb[...]

    xu = xeb[:, :n + 2 * w]
    xm = xeb[:, w:n + 3 * w]
    xd = xeb[:, 2 * w:]
    s = xu + 2.0 * xm + xd
    t = xd - xu

    ta = xt[0][3:4, 6:8, :].reshape(1, 2 * w) * tfac
    ba = xb[0][3:4, 0:2, :].reshape(1, 2 * w) * bfac
    ae = jnp.concatenate([ta, xc[0][3:4].reshape(1, n), ba], axis=1)
    pmf = jnp.maximum(jnp.maximum(ae[:, :n + 2 * w], ae[:, w:n + 3 * w]),
                      ae[:, 2 * w:])
    ml = mle[...]
    mr = mre[...]
    pooled = jnp.maximum(
        jnp.maximum(pmf[:, w + 1:w + 1 + n] * ml, pmf[:, w - 1:w - 1 + n] * mr),
        pmf[:, w:w + n])
    act = jnp.where((pooled > 0.1) & (fm[0].reshape(1, n) != 0), 1.0, 0.0)

    us = []
    n2 = n // 12
    for half in range(12):
        o = half * n2
        gx = (s[:, w + 1 + o:w + 1 + o + n2] * mlb[:, o:o + n2]
              - s[:, w - 1 + o:w - 1 + o + n2] * mrb[:, o:o + n2])
        gy = (t[:, w + 1 + o:w + 1 + o + n2] * mlb[:, o:o + n2]
              + t[:, w - 1 + o:w - 1 + o + n2] * mrb[:, o:o + n2]
              + 2.0 * t[:, w + o:w + o + n2])
        mid = xeb[:, 2 * w + o:2 * w + o + n2]
        f = jnp.concatenate([mid, gx, gy], axis=0)
        h1 = jnp.dot(w1[...], f, preferred_element_type=jnp.float32)
        h1 = jnp.maximum(h1.astype(jnp.bfloat16) + b1[...], 0)
        h2 = jnp.dot(w2[...], h1, preferred_element_type=jnp.float32)
        h2 = jnp.maximum(h2.astype(jnp.bfloat16) + b2[...], 0)
        us.append(jnp.dot(w3[...], h2, preferred_element_type=jnp.float32))
    u = jnp.concatenate(us, axis=1) + b3[...]
    out[0] = (u * act).reshape(_CH, _HB, w)


def kernel(x, fire_mask, W1, b1, W2, b2, W3, b3):
    B, C, H, W = x.shape
    nh = H // _HB
    n = _HB * W
    nhb = H // 8
    wcol = jnp.arange(n, dtype=jnp.int32) % W
    mle = (wcol != W - 1).astype(jnp.float32).reshape(1, n)
    mre = (wcol != 0).astype(jnp.float32).reshape(1, n)
    w1b = W1.astype(jnp.bfloat16)
    w2b = W2.astype(jnp.bfloat16)
    w3b = W3.astype(jnp.bfloat16)
    b1c = b1.astype(jnp.bfloat16).reshape(_EMB, 1)
    b2c = b2.astype(jnp.bfloat16).reshape(_EMB, 1)
    b3c = b3.reshape(_CH, 1)
    k = _HB // 8

    def const_spec(shape):
        return pl.BlockSpec(shape, lambda b, h: tuple(0 for _ in shape))

    return pl.pallas_call(
        _fused_kernel,
        grid=(B, nh),
        in_specs=[
            pl.BlockSpec((1, C, 8, W),
                         lambda b, h: (b, 0, jnp.maximum(k * h - 1, 0), 0)),
            pl.BlockSpec((1, C, _HB, W), lambda b, h: (b, 0, h, 0)),
            pl.BlockSpec((1, C, 8, W),
                         lambda b, h: (b, 0, jnp.minimum(k * (h + 1), nhb - 1), 0)),
            pl.BlockSpec((1, 1, _HB, W), lambda b, h: (b, 0, h, 0)),
            const_spec((1, n)),
            const_spec((1, n)),
            const_spec((1, n)),
            const_spec((1, n)),
            const_spec((_EMB, 3 * _CH)),
            const_spec((_EMB, 1)),
            const_spec((_EMB, _EMB)),
            const_spec((_EMB, 1)),
            const_spec((_CH, _EMB)),
            const_spec((_CH, 1)),
        ],
        out_specs=pl.BlockSpec((1, C, _HB, W), lambda b, h: (b, 0, h, 0)),
        out_shape=jax.ShapeDtypeStruct((B, C, H, W), jnp.float32),
        compiler_params=pltpu.CompilerParams(
            dimension_semantics=("parallel", "parallel")),
    )(x, x, x, fire_mask, mle, mre, mle.astype(jnp.bfloat16),
      mre.astype(jnp.bfloat16), w1b, b1c, w2b, b2c, w3b, b3c)
